# Initial kernel scaffold; baseline (speedup 1.0000x reference)
import jax
import jax.numpy as jnp
from jax.experimental import pallas as pl


def kernel(inputs, table):
    raise NotImplementedError("write your pallas kernel here")


# speedup vs baseline: 5.0821x; 5.0821x over previous
import functools

import jax
import jax.numpy as jnp
from jax import lax
from jax.experimental import pallas as pl
from jax.experimental.pallas import tpu as pltpu
from jax.experimental.pallas import tpu_sc as plsc

NUM_TABLE_ROWS = 100000
D = 64
HW = 256
K = 3
IDX_MINOR = 128
IDX_CHUNKS = (K * HW) // IDX_MINOR
OUT_PAD = 257


def _sc_bow_embed(idx, table, batch):
    nw = 32
    batches_per_worker = batch // nw

    mesh = plsc.VectorSubcoreMesh(core_axis_name="c", subcore_axis_name="s")

    @functools.partial(
        pl.kernel,
        out_type=jax.ShapeDtypeStruct((batch, D, HW), jnp.float32),
        mesh=mesh,
        compiler_params=pltpu.CompilerParams(
            needs_layout_passes=False, use_tc_tiling_on_sc=False),
        scratch_types=[
            pltpu.VMEM((IDX_CHUNKS, IDX_MINOR), jnp.int32),
            pltpu.VMEM((K * HW, D), jnp.float32),
            pltpu.VMEM((D, OUT_PAD), jnp.float32),
            pltpu.SemaphoreType.DMA,
        ],
    )
    def body(idx_hbm, table_hbm, out_hbm, idx_v, rows_v, out_t, sem):
        wid = lax.axis_index("s") * 2 + lax.axis_index("c")
        lane = lax.iota(jnp.int32, 16)
        third = jnp.float32(1.0 / 3.0)
        e_rows = [c * 16 + lane for c in range(4)]

        def batch_body(i, _):
            b = wid * batches_per_worker + i
            pltpu.sync_copy(idx_hbm.at[b], idx_v)
            copies = [
                pltpu.async_copy(
                    table_hbm.at[idx_v.at[j]],
                    rows_v.at[pl.ds(j * IDX_MINOR, IDX_MINOR)],
                    sem,
                )
                for j in range(IDX_CHUNKS)
            ]
            for cp in copies:
                cp.wait()

            def hw_body(hw, _):
                col = jnp.zeros((16,), jnp.int32) + hw
                for c in range(4):
                    sl = pl.ds(c * 16, 16)
                    v = (rows_v[hw, sl] + rows_v[HW + hw, sl]
                         + rows_v[2 * HW + hw, sl]) * third
                    plsc.store_scatter(out_t, [e_rows[c], col], v)
                return 0

            lax.fori_loop(0, HW, hw_body, 0)
            pltpu.sync_copy(out_t.at[:, pl.ds(0, HW)], out_hbm.at[b])
            return 0

        lax.fori_loop(0, batches_per_worker, batch_body, 0)

    return body(idx, table)


def kernel(inputs, table):
    b, k, h, w = inputs.shape
    idx = inputs.reshape(b, IDX_CHUNKS, IDX_MINOR)
    out = _sc_bow_embed(idx, table, b)
    return out.reshape(b, D, h, w)


# speedup vs baseline: 5.9911x; 1.1789x over previous
import functools

import jax
import jax.numpy as jnp
from jax import lax
from jax.experimental import pallas as pl
from jax.experimental.pallas import tpu as pltpu
from jax.experimental.pallas import tpu_sc as plsc

D = 64
HW = 256
K = 3
IDX_MINOR = 128
IDX_CHUNKS = (K * HW) // IDX_MINOR
OUT_PAD = 257
NW = 32


def _sc_bow_embed(idx, table, batch):
    nb = batch // NW

    mesh = plsc.VectorSubcoreMesh(core_axis_name="c", subcore_axis_name="s")

    @functools.partial(
        pl.kernel,
        out_type=jax.ShapeDtypeStruct((batch, D, HW), jnp.float32),
        mesh=mesh,
        compiler_params=pltpu.CompilerParams(
            needs_layout_passes=False, use_tc_tiling_on_sc=False),
        scratch_types=[
            pltpu.VMEM((2, IDX_CHUNKS, IDX_MINOR), jnp.int32),
            pltpu.VMEM((2, K * HW, D), jnp.float32),
            pltpu.VMEM((D, OUT_PAD), jnp.float32),
            pltpu.SemaphoreType.DMA,
            pltpu.SemaphoreType.DMA,
            pltpu.SemaphoreType.DMA,
        ],
    )
    def body(idx_hbm, table_hbm, out_hbm, idx_v, rows_v, out_t, g0, g1, osem):
        wid = lax.axis_index("s") * 2 + lax.axis_index("c")
        b0 = wid * nb
        lane = lax.iota(jnp.int32, 16)
        third = jnp.float32(1.0 / 3.0)
        e_rows = [c * 16 + lane for c in range(4)]
        gsem = (g0, g1)

        def fire(i, buf):
            for j in range(IDX_CHUNKS):
                pltpu.async_copy(
                    table_hbm.at[idx_v.at[buf, j]],
                    rows_v.at[buf, pl.ds(j * IDX_MINOR, IDX_MINOR)],
                    gsem[buf],
                )

        def drain(buf):
            for j in range(IDX_CHUNKS):
                pltpu.make_async_copy(
                    table_hbm.at[idx_v.at[buf, j]],
                    rows_v.at[buf, pl.ds(j * IDX_MINOR, IDX_MINOR)],
                    gsem[buf],
                ).wait()

        def out_start(i):
            pltpu.async_copy(
                out_t.at[:, pl.ds(0, HW)], out_hbm.at[b0 + i], osem)

        def out_wait(i):
            pltpu.make_async_copy(
                out_t.at[:, pl.ds(0, HW)], out_hbm.at[b0 + i], osem).wait()

        def compute(buf):
            rv = rows_v.at[buf]

            def hw_body(hw, _):
                col = jnp.zeros((16,), jnp.int32) + hw
                for c in range(4):
                    sl = pl.ds(c * 16, 16)
                    v = (rv[hw, sl] + rv[HW + hw, sl]
                         + rv[2 * HW + hw, sl]) * third
                    plsc.store_scatter(out_t, [e_rows[c], col], v)
                return 0

            lax.fori_loop(0, HW, hw_body, 0)

        pltpu.sync_copy(idx_hbm.at[b0], idx_v.at[0])
        fire(0, 0)

        def pair_body(p, _):
            i = p * 2
            for par in (0, 1):
                ii = i + par
                nxt = ii + 1

                @pl.when(nxt < nb)
                def _():
                    pltpu.sync_copy(idx_hbm.at[b0 + nxt], idx_v.at[1 - par])
                    fire(nxt, 1 - par)

                drain(par)

                @pl.when(ii > 0)
                def _():
                    out_wait(ii - 1)

                compute(par)
                out_start(ii)
            return 0

        lax.fori_loop(0, nb // 2, pair_body, 0)
        out_wait(nb - 1)

    return body(idx, table)


def kernel(inputs, table):
    b, k, h, w = inputs.shape
    idx = inputs.reshape(b, IDX_CHUNKS, IDX_MINOR)
    out = _sc_bow_embed(idx, table, b)
    return out.reshape(b, D, h, w)
